# Initial kernel scaffold; baseline (speedup 1.0000x reference)
import jax
import jax.numpy as jnp
from jax.experimental import pallas as pl


def kernel(inputs, base_W, base_b, W_out, b_out, stage_bias):
    raise NotImplementedError("write your pallas kernel here")


# speedup vs baseline: 2.7572x; 2.7572x over previous
import functools

import jax
import jax.numpy as jnp
from jax import lax
from jax.experimental import pallas as pl
from jax.experimental.pallas import tpu as pltpu

NUM_STAGES = 4
OPTIONS = 512
CODE_DIM = 32
HIDDEN = 64


def _dot_t(a, b):
    return lax.dot_general(
        a, b, (((0,), (0,)), ((), ())), preferred_element_type=jnp.float32
    )


def _encoder_kernel(xT_ref, bw_ref, bb_ref, wout_ref, bout_ref, sb_ref,
                    enc_ref, cur_ref, loss_ref):
    xT = xT_ref[...]
    T = xT.shape[1]
    cur = jnp.zeros((CODE_DIM, T), jnp.float32)
    for s in range(NUM_STAGES):
        h = jnp.maximum(_dot_t(bw_ref[...], cur) + bb_ref[...], 0.0)
        layer = _dot_t(wout_ref[s], h)
        l3 = layer.reshape(OPTIONS, CODE_DIM, T)
        n3 = cur[None, :, :] + (sb_ref[s][:, :, None] + (l3 + bout_ref[s][:, :, None]))
        d3 = n3 - xT[None, :, :]
        ls = jnp.mean(d3 * d3, axis=1)
        loss_ref[s] = ls
        idx = jnp.argmin(ls, axis=0)
        enc_ref[s] = idx
        onehot = lax.broadcasted_iota(jnp.int32, (OPTIONS, T), 0) == idx[None, :]
        cur = jnp.sum(jnp.where(onehot[:, None, :], n3, 0.0), axis=0)
    cur_ref[...] = cur


@jax.jit
def kernel(inputs, base_W, base_b, W_out, b_out, stage_bias):
    n = inputs.shape[0]
    T = 128
    xT = inputs.T
    bb = base_b.reshape(HIDDEN, 1)
    bout = b_out.reshape(NUM_STAGES, OPTIONS, CODE_DIM)

    grid = (n // T,)
    enc_t, cur_t, loss_t = pl.pallas_call(
        _encoder_kernel,
        grid=grid,
        in_specs=[
            pl.BlockSpec((CODE_DIM, T), lambda t: (0, t)),
            pl.BlockSpec((CODE_DIM, HIDDEN), lambda t: (0, 0)),
            pl.BlockSpec((HIDDEN, 1), lambda t: (0, 0)),
            pl.BlockSpec((NUM_STAGES, HIDDEN, OPTIONS * CODE_DIM), lambda t: (0, 0, 0)),
            pl.BlockSpec((NUM_STAGES, OPTIONS, CODE_DIM), lambda t: (0, 0, 0)),
            pl.BlockSpec((NUM_STAGES, OPTIONS, CODE_DIM), lambda t: (0, 0, 0)),
        ],
        out_specs=[
            pl.BlockSpec((NUM_STAGES, T), lambda t: (0, t)),
            pl.BlockSpec((CODE_DIM, T), lambda t: (0, t)),
            pl.BlockSpec((NUM_STAGES, OPTIONS, T), lambda t: (0, 0, t)),
        ],
        out_shape=[
            jax.ShapeDtypeStruct((NUM_STAGES, n), jnp.int32),
            jax.ShapeDtypeStruct((CODE_DIM, n), jnp.float32),
            jax.ShapeDtypeStruct((NUM_STAGES, OPTIONS, n), jnp.float32),
        ],
        compiler_params=pltpu.CompilerParams(
            dimension_semantics=("arbitrary",),
            vmem_limit_bytes=100 * 1024 * 1024,
        ),
    )(xT, base_W, bb, W_out, bout, stage_bias)

    encodings = enc_t.T
    cur = cur_t.T
    losses = jnp.transpose(loss_t, (2, 0, 1))
    return (encodings, cur, losses)
